# baseline (speedup 1.0000x reference)
import functools

import jax
import jax.numpy as jnp
from jax import lax
from jax.experimental import pallas as pl
from jax.experimental.pallas import tpu as pltpu
from jax.experimental.pallas import tpu_sc as plsc

NE = 64
NK = 32
D = 2048
DFF = 768
NTOK = 2048
TOPK = 2

EPAD = 128
TB = 256
T = 128
NB = 64
PPAD = NB * T

NEG = -1e30


def _router_body(x_ref, gk_ref, bias_ref, o2n_ref,
                 exp_ref, rank_ref, w_ref, offs_ref, be_ref, nb_ref,
                 carry_ref):
    s = pl.program_id(0)
    t = pl.program_id(1)

    @pl.when((s == 0) & (t == 0))
    def _():
        carry_ref[...] = jnp.zeros_like(carry_ref)

    xb = x_ref[...]
    gk = gk_ref[...]
    logits = lax.dot_general(xb, gk, (((1,), (1,)), ((), ())),
                             preferred_element_type=jnp.float32)
    logits = logits + bias_ref[...]

    lane = lax.broadcasted_iota(jnp.int32, (TB, EPAD), 1)
    m0 = jnp.max(logits, axis=1, keepdims=True)
    a0 = jnp.min(jnp.where(logits == m0, lane, EPAD), axis=1, keepdims=True)
    masked = jnp.where(lane == a0, NEG, logits)
    m1 = jnp.max(masked, axis=1, keepdims=True)
    a1 = jnp.min(jnp.where(masked == m1, lane, EPAD), axis=1, keepdims=True)

    sel0 = s == 0
    a_sel = jnp.where(sel0, a0, a1)
    l_sel = jnp.where(sel0, m0, m1)
    l_oth = jnp.where(sel0, m1, m0)
    w = jax.nn.sigmoid(l_sel - l_oth)

    oh_old = lane == a_sel
    e_new = jnp.sum(jnp.where(oh_old, o2n_ref[...], 0), axis=1, keepdims=True)

    onew = (lane == e_new).astype(jnp.float32)
    row = lax.broadcasted_iota(jnp.int32, (TB, TB), 0)
    col = lax.broadcasted_iota(jnp.int32, (TB, TB), 1)
    ltri = (row > col).astype(jnp.float32)
    csum = jnp.dot(ltri, onew, preferred_element_type=jnp.float32)
    carry = carry_ref[...]
    rank = (jnp.sum(onew * csum, axis=1, keepdims=True)
            + jnp.sum(onew * carry, axis=1, keepdims=True))
    carry_ref[...] = carry + jnp.sum(onew, axis=0, keepdims=True)

    exp_ref[0] = e_new
    rank_ref[0] = rank.astype(jnp.int32)
    w_ref[0] = w

    cnt = carry_ref[...]
    pad_cnt = jnp.ceil(cnt / T) * T
    rowm = lax.broadcasted_iota(jnp.int32, (EPAD, EPAD), 0)
    colm = lax.broadcasted_iota(jnp.int32, (EPAD, EPAD), 1)
    um = (rowm < colm).astype(jnp.float32)
    offs = jnp.dot(pad_cnt, um, preferred_element_type=jnp.float32)
    offs_end = offs + pad_cnt
    total = jnp.sum(pad_cnt)
    nbv = (total / T).astype(jnp.int32)
    bv = (lax.broadcasted_iota(jnp.int32, (NB, EPAD), 0) * T).astype(jnp.float32)
    be = jnp.sum((offs_end <= bv).astype(jnp.int32), axis=1, keepdims=True)
    be = jnp.minimum(be, NK - 1)
    offs_ref[...] = offs.astype(jnp.int32).reshape(1, 1, EPAD)
    be_ref[0] = be
    nb_ref[...] = jnp.full((1, 1, 8), nbv, jnp.int32)


def _run_router(x2d, gkp, bias_row, o2n_row):
    nsteps = NTOK // TB
    out = pl.pallas_call(
        _router_body,
        grid=(TOPK, nsteps),
        in_specs=[
            pl.BlockSpec((TB, D), lambda s, t: (t, 0)),
            pl.BlockSpec((EPAD, D), lambda s, t: (0, 0)),
            pl.BlockSpec((1, EPAD), lambda s, t: (0, 0)),
            pl.BlockSpec((1, EPAD), lambda s, t: (0, 0)),
        ],
        out_specs=[
            pl.BlockSpec((1, TB, 1), lambda s, t: (s * 8 + t, 0, 0)),
            pl.BlockSpec((1, TB, 1), lambda s, t: (s * 8 + t, 0, 0)),
            pl.BlockSpec((1, TB, 1), lambda s, t: (s * 8 + t, 0, 0)),
            pl.BlockSpec((1, 1, EPAD), lambda s, t: (0, 0, 0)),
            pl.BlockSpec((1, NB, 1), lambda s, t: (0, 0, 0)),
            pl.BlockSpec((1, 1, 8), lambda s, t: (0, 0, 0)),
        ],
        out_shape=[
            jax.ShapeDtypeStruct((TOPK * nsteps, TB, 1), jnp.int32),
            jax.ShapeDtypeStruct((TOPK * nsteps, TB, 1), jnp.int32),
            jax.ShapeDtypeStruct((TOPK * nsteps, TB, 1), jnp.float32),
            jax.ShapeDtypeStruct((1, 1, EPAD), jnp.int32),
            jax.ShapeDtypeStruct((1, NB, 1), jnp.int32),
            jax.ShapeDtypeStruct((1, 1, 8), jnp.int32),
        ],
        scratch_shapes=[pltpu.VMEM((1, EPAD), jnp.float32)],
        interpret=False,
    )(x2d, gkp, bias_row, o2n_row)
    exp_flat = out[0].reshape(TOPK * NTOK)
    rank_flat = out[1].reshape(TOPK * NTOK)
    w_flat = out[2].reshape(TOPK * NTOK)
    offs = out[3].reshape(EPAD)[:NK]
    be = out[4].reshape(NB)
    nbv8 = out[5].reshape(8)
    return exp_flat, rank_flat, w_flat, offs, be, nbv8


NASSIGN = TOPK * NTOK
_SC_MESH = dict(core_axis_name="c", subcore_axis_name="s",
                num_cores=2, num_subcores=16)
NW = 32
ROWS_W = PPAD // NW
GCH = 32
TOK_W = NTOK // NW


def _scatter_body(exp_hbm, rank_hbm, w_hbm, offs_hbm,
                  tok_hbm, ws_hbm, pos_hbm,
                  exp_v, rank_v, w_v, offs_v, tok_v, ws_v, pos_v):
    cid = lax.axis_index("c")
    sid = lax.axis_index("s")

    @pl.when((cid == 0) & (sid == 0))
    def _():
        pltpu.sync_copy(exp_hbm, exp_v)
        pltpu.sync_copy(rank_hbm, rank_v)
        pltpu.sync_copy(w_hbm, w_v)
        pltpu.sync_copy(offs_hbm, offs_v)
        zi = jnp.zeros((16,), jnp.int32)
        zf = jnp.zeros((16,), jnp.float32)

        def zbody(i, carry):
            tok_v[pl.ds(i * 16, 16)] = zi
            ws_v[pl.ds(i * 16, 16)] = zf
            return carry

        lax.fori_loop(0, PPAD // 16, zbody, 0)
        iota = lax.iota(jnp.int32, 16)

        def body(ch, carry):
            base = ch * 16
            e = exp_v[pl.ds(base, 16)]
            r = rank_v[pl.ds(base, 16)]
            wv = w_v[pl.ds(base, 16)]
            p = plsc.load_gather(offs_v, [e]) + r
            i_vec = base + iota
            tkn = jnp.where(i_vec >= NTOK, i_vec - NTOK, i_vec)
            plsc.store_scatter(tok_v, [p], tkn)
            plsc.store_scatter(ws_v, [p], wv)
            pos_v[pl.ds(base, 16)] = p
            return carry

        lax.fori_loop(0, NASSIGN // 16, body, 0)
        pltpu.sync_copy(tok_v, tok_hbm)
        pltpu.sync_copy(ws_v, ws_hbm)
        pltpu.sync_copy(pos_v, pos_hbm)


def _scatter_meta(exp_flat, rank_flat, w_flat, offs):
    fn = pl.kernel(
        _scatter_body,
        out_type=[
            jax.ShapeDtypeStruct((PPAD,), jnp.int32),
            jax.ShapeDtypeStruct((PPAD,), jnp.float32),
            jax.ShapeDtypeStruct((NASSIGN,), jnp.int32),
        ],
        mesh=plsc.VectorSubcoreMesh(**_SC_MESH),
        compiler_params=pltpu.CompilerParams(needs_layout_passes=False),
        scratch_types=[
            pltpu.VMEM((NASSIGN,), jnp.int32),
            pltpu.VMEM((NASSIGN,), jnp.int32),
            pltpu.VMEM((NASSIGN,), jnp.float32),
            pltpu.VMEM((NK,), jnp.int32),
            pltpu.VMEM((PPAD,), jnp.int32),
            pltpu.VMEM((PPAD,), jnp.float32),
            pltpu.VMEM((NASSIGN,), jnp.int32),
        ],
    )
    return fn(exp_flat, rank_flat, w_flat, offs)


def _combine_body(ys_hbm, pos_hbm, out_hbm,
                  i0_v, i1_v, r0_v, r1_v, sem0, sem1):
    wid = lax.axis_index("c") * 16 + lax.axis_index("s")
    base = wid * TOK_W

    def body(ch, carry):
        o = base + ch * 16
        pltpu.sync_copy(pos_hbm.at[pl.ds(o, 16)], i0_v)
        pltpu.sync_copy(pos_hbm.at[pl.ds(NTOK + o, 16)], i1_v)
        d0 = pltpu.async_copy(ys_hbm.at[i0_v], r0_v, sem0)
        d1 = pltpu.async_copy(ys_hbm.at[i1_v], r1_v, sem1)
        d0.wait()
        d1.wait()

        def add_body(k, c2):
            col = k * 16
            for rr in range(16):
                r0_v[rr, pl.ds(col, 16)] = (r0_v[rr, pl.ds(col, 16)]
                                            + r1_v[rr, pl.ds(col, 16)])
            return c2

        lax.fori_loop(0, D // 16, add_body, 0)
        pltpu.sync_copy(r0_v, out_hbm.at[pl.ds(o, 16)])
        return carry

    lax.fori_loop(0, TOK_W // 16, body, 0)


def _combine(ys, pos):
    fn = pl.kernel(
        _combine_body,
        out_type=[jax.ShapeDtypeStruct((NTOK, D), jnp.float32)],
        mesh=plsc.VectorSubcoreMesh(**_SC_MESH),
        compiler_params=pltpu.CompilerParams(needs_layout_passes=False),
        scratch_types=[
            pltpu.VMEM((16,), jnp.int32),
            pltpu.VMEM((16,), jnp.int32),
            pltpu.VMEM((16, D), jnp.float32),
            pltpu.VMEM((16, D), jnp.float32),
            pltpu.SemaphoreType.DMA,
            pltpu.SemaphoreType.DMA,
        ],
    )
    return fn(ys, pos)[0]


def _mlp_body(be_ref, nb_ref, x_ref, tok_ref, gu_ref, dp_ref, w_ref, out_ref):
    b = pl.program_id(0)
    nbv = nb_ref[0]

    @pl.when(b == 0)
    def _():
        out_ref[...] = jnp.zeros_like(out_ref)

    @pl.when(b < nbv)
    def _():
        tok_row = tok_ref[0]
        sub = lax.broadcasted_iota(jnp.int32, (NTOK, T), 0)
        hit = sub == tok_row
        permt = hit.astype(jnp.bfloat16)
        xb = lax.dot_general(permt, x_ref[...], (((0,), (0,)), ((), ())),
                             preferred_element_type=jnp.float32)
        xb = xb.astype(jnp.bfloat16)
        gu = gu_ref[0].astype(jnp.bfloat16)
        g_up = lax.dot_general(xb, gu, (((1,), (1,)), ((), ())),
                               preferred_element_type=jnp.float32)
        gate = g_up[:, :DFF]
        up = g_up[:, DFF:]
        h = gate * jax.nn.sigmoid(gate) * up
        dp = dp_ref[0].astype(jnp.bfloat16)
        y = lax.dot_general(h.astype(jnp.bfloat16), dp,
                            (((1,), (1,)), ((), ())),
                            preferred_element_type=jnp.float32)
        permw = jnp.where(hit, w_ref[0], 0.0).astype(jnp.bfloat16)
        delta = lax.dot_general(permw, y.astype(jnp.bfloat16),
                                (((1,), (0,)), ((), ())),
                                preferred_element_type=jnp.float32)
        out_ref[...] += delta


def _run_mlp(x_bf, tok3, gate_up_proj, down_proj, ws3, be, nbv):
    def _clamped(b, be_s, nb_s):
        return jnp.where(b < nb_s[0], b, nb_s[0] - 1)

    grid_spec = pltpu.PrefetchScalarGridSpec(
        num_scalar_prefetch=2,
        grid=(NB,),
        in_specs=[
            pl.BlockSpec((NTOK, D), lambda b, be_s, nb_s: (0, 0)),
            pl.BlockSpec((1, 1, T), lambda b, be_s, nb_s: (_clamped(b, be_s, nb_s), 0, 0)),
            pl.BlockSpec((1, 2 * DFF, D),
                         lambda b, be_s, nb_s: (be_s[_clamped(b, be_s, nb_s)], 0, 0)),
            pl.BlockSpec((1, D, DFF),
                         lambda b, be_s, nb_s: (be_s[_clamped(b, be_s, nb_s)], 0, 0)),
            pl.BlockSpec((1, 1, T), lambda b, be_s, nb_s: (_clamped(b, be_s, nb_s), 0, 0)),
        ],
        out_specs=pl.BlockSpec((NTOK, D), lambda b, be_s, nb_s: (0, 0)),
    )
    return pl.pallas_call(
        _mlp_body,
        grid_spec=grid_spec,
        out_shape=jax.ShapeDtypeStruct((NTOK, D), jnp.float32),
        compiler_params=pltpu.CompilerParams(
            vmem_limit_bytes=120 * 1024 * 1024),
        interpret=False,
    )(be, nbv, x_bf, tok3, gate_up_proj, down_proj, ws3)


def kernel(hidden_states, gate_weight, gate_up_proj, down_proj, old_to_new):
    bsz, seq, _ = hidden_states.shape
    x2d = hidden_states.reshape(NTOK, D)

    gkp = jnp.zeros((EPAD, D), jnp.float32).at[:NE].set(gate_weight)
    pruned = old_to_new < 0
    bias_row = jnp.where(
        jnp.concatenate([pruned, jnp.ones((EPAD - NE,), bool)]),
        jnp.float32(NEG), jnp.float32(0.0)).reshape(1, EPAD)
    o2n_row = jnp.concatenate(
        [old_to_new, jnp.zeros((EPAD - NE,), jnp.int32)]).reshape(1, EPAD)

    exp_flat, rank_flat, w_flat, offs, be, nbv8 = _run_router(
        x2d, gkp, bias_row, o2n_row)

    tok_sorted, w_sorted, pos = _scatter_meta(exp_flat, rank_flat, w_flat, offs)

    x_bf = x2d.astype(jnp.bfloat16)
    tok3 = tok_sorted.reshape(NB, 1, T)
    ws3 = w_sorted.reshape(NB, 1, T)
    out = _run_mlp(x_bf, tok3, gate_up_proj, down_proj, ws3, be, nbv8[:1])
    return out.reshape(bsz, seq, D)


# speedup vs baseline: 1.2450x; 1.2450x over previous
import functools

import jax
import jax.numpy as jnp
from jax import lax
from jax.experimental import pallas as pl
from jax.experimental.pallas import tpu as pltpu
from jax.experimental.pallas import tpu_sc as plsc

NE = 64
NK = 32
D = 2048
DFF = 768
NTOK = 2048
TOPK = 2

EPAD = 128
TB = 256
T = 256
NB = 48
PPAD = NB * T

NEG = -1e30


def _router_body(x_ref, gk_ref, bias_ref, o2n_ref,
                 exp_ref, rank_ref, w_ref, offs_ref, be_ref, nb_ref,
                 carry_ref):
    t = pl.program_id(0)

    @pl.when(t == 0)
    def _():
        carry_ref[...] = jnp.zeros_like(carry_ref)

    xb = x_ref[...]
    gk = gk_ref[...]
    logits = lax.dot_general(xb, gk, (((1,), (1,)), ((), ())),
                             preferred_element_type=jnp.float32)
    logits = logits + bias_ref[...]

    lane = lax.broadcasted_iota(jnp.int32, (TB, EPAD), 1)
    m0 = jnp.max(logits, axis=1, keepdims=True)
    a0 = jnp.min(jnp.where(logits == m0, lane, EPAD), axis=1, keepdims=True)
    masked = jnp.where(lane == a0, NEG, logits)
    m1 = jnp.max(masked, axis=1, keepdims=True)
    a1 = jnp.min(jnp.where(masked == m1, lane, EPAD), axis=1, keepdims=True)

    w0 = jax.nn.sigmoid(m0 - m1)
    w1 = jax.nn.sigmoid(m1 - m0)

    o2n = o2n_ref[...]
    e0 = jnp.sum(jnp.where(lane == a0, o2n, 0), axis=1, keepdims=True)
    e1 = jnp.sum(jnp.where(lane == a1, o2n, 0), axis=1, keepdims=True)

    oh0 = (lane == e0).astype(jnp.float32)
    oh1 = (lane == e1).astype(jnp.float32)
    both = oh0 + oh1
    row = lax.broadcasted_iota(jnp.int32, (TB, TB), 0)
    col = lax.broadcasted_iota(jnp.int32, (TB, TB), 1)
    ltri = (row > col).astype(jnp.float32)
    csum = jnp.dot(ltri, both, preferred_element_type=jnp.float32)
    carry = carry_ref[...]
    base_cnt = csum + carry
    rank0 = jnp.sum(oh0 * base_cnt, axis=1, keepdims=True)
    rank1 = jnp.sum(oh1 * base_cnt, axis=1, keepdims=True)
    carry_ref[...] = carry + jnp.sum(both, axis=0, keepdims=True)

    exp_ref[0] = jnp.concatenate([e0, e1], axis=1)
    rank_ref[0] = jnp.concatenate([rank0, rank1], axis=1).astype(jnp.int32)
    w_ref[0] = jnp.concatenate([w0, w1], axis=1)

    cnt = carry_ref[...]
    pad_cnt = jnp.ceil(cnt / T) * T
    rowm = lax.broadcasted_iota(jnp.int32, (EPAD, EPAD), 0)
    colm = lax.broadcasted_iota(jnp.int32, (EPAD, EPAD), 1)
    um = (rowm < colm).astype(jnp.float32)
    offs = jnp.dot(pad_cnt, um, preferred_element_type=jnp.float32)
    offs_end = offs + pad_cnt
    total = jnp.sum(pad_cnt)
    nbv = (total / T).astype(jnp.int32)
    bv = (lax.broadcasted_iota(jnp.int32, (NB, EPAD), 0) * T).astype(jnp.float32)
    be = jnp.sum((offs_end <= bv).astype(jnp.int32), axis=1, keepdims=True)
    be = jnp.minimum(be, NK - 1)
    offs_ref[...] = offs.astype(jnp.int32).reshape(1, 1, EPAD)
    be_ref[0] = be
    nb_ref[...] = jnp.full((1, 1, 8), nbv, jnp.int32)


def _run_router(x2d, gkp, bias_row, o2n_row):
    nsteps = NTOK // TB
    out = pl.pallas_call(
        _router_body,
        grid=(nsteps,),
        in_specs=[
            pl.BlockSpec((TB, D), lambda t: (t, 0)),
            pl.BlockSpec((EPAD, D), lambda t: (0, 0)),
            pl.BlockSpec((1, EPAD), lambda t: (0, 0)),
            pl.BlockSpec((1, EPAD), lambda t: (0, 0)),
        ],
        out_specs=[
            pl.BlockSpec((1, TB, TOPK), lambda t: (t, 0, 0)),
            pl.BlockSpec((1, TB, TOPK), lambda t: (t, 0, 0)),
            pl.BlockSpec((1, TB, TOPK), lambda t: (t, 0, 0)),
            pl.BlockSpec((1, 1, EPAD), lambda t: (0, 0, 0)),
            pl.BlockSpec((1, NB, 1), lambda t: (0, 0, 0)),
            pl.BlockSpec((1, 1, 8), lambda t: (0, 0, 0)),
        ],
        out_shape=[
            jax.ShapeDtypeStruct((nsteps, TB, TOPK), jnp.int32),
            jax.ShapeDtypeStruct((nsteps, TB, TOPK), jnp.int32),
            jax.ShapeDtypeStruct((nsteps, TB, TOPK), jnp.float32),
            jax.ShapeDtypeStruct((1, 1, EPAD), jnp.int32),
            jax.ShapeDtypeStruct((1, NB, 1), jnp.int32),
            jax.ShapeDtypeStruct((1, 1, 8), jnp.int32),
        ],
        scratch_shapes=[pltpu.VMEM((1, EPAD), jnp.float32)],
        interpret=False,
    )(x2d, gkp, bias_row, o2n_row)
    exp_flat = out[0].reshape(TOPK * NTOK)
    rank_flat = out[1].reshape(TOPK * NTOK)
    w_flat = out[2].reshape(TOPK * NTOK)
    offs = out[3].reshape(EPAD)[:NK]
    be = out[4].reshape(NB)
    nbv8 = out[5].reshape(8)
    return exp_flat, rank_flat, w_flat, offs, be, nbv8


NASSIGN = TOPK * NTOK
_SC_MESH = dict(core_axis_name="c", subcore_axis_name="s",
                num_cores=2, num_subcores=16)
NW = 32
ROWS_W = PPAD // NW
GCH = 32
TOK_W = NTOK // NW


def _scatter_body(exp_hbm, rank_hbm, w_hbm, offs_hbm,
                  tok_hbm, ws_hbm, pos_hbm,
                  exp_v, rank_v, w_v, offs_v, tok_v, ws_v, pos_v):
    cid = lax.axis_index("c")
    sid = lax.axis_index("s")

    @pl.when((cid == 0) & (sid == 0))
    def _():
        pltpu.sync_copy(exp_hbm, exp_v)
        pltpu.sync_copy(rank_hbm, rank_v)
        pltpu.sync_copy(w_hbm, w_v)
        pltpu.sync_copy(offs_hbm, offs_v)
        zi = jnp.zeros((16,), jnp.int32)
        zf = jnp.zeros((16,), jnp.float32)

        def zbody(i, carry):
            tok_v[pl.ds(i * 16, 16)] = zi
            ws_v[pl.ds(i * 16, 16)] = zf
            return carry

        lax.fori_loop(0, PPAD // 16, zbody, 0)
        iota = lax.iota(jnp.int32, 16)

        def body(ch, carry):
            base = ch * 16
            e = exp_v[pl.ds(base, 16)]
            r = rank_v[pl.ds(base, 16)]
            wv = w_v[pl.ds(base, 16)]
            p = plsc.load_gather(offs_v, [e]) + r
            i_vec = base + iota
            tkn = lax.shift_right_logical(i_vec, 1)
            plsc.store_scatter(tok_v, [p], tkn)
            plsc.store_scatter(ws_v, [p], wv)
            pos_v[pl.ds(base, 16)] = p
            return carry

        lax.fori_loop(0, NASSIGN // 16, body, 0)
        pltpu.sync_copy(tok_v, tok_hbm)
        pltpu.sync_copy(ws_v, ws_hbm)
        pltpu.sync_copy(pos_v, pos_hbm)


def _scatter_meta(exp_flat, rank_flat, w_flat, offs):
    fn = pl.kernel(
        _scatter_body,
        out_type=[
            jax.ShapeDtypeStruct((PPAD,), jnp.int32),
            jax.ShapeDtypeStruct((PPAD,), jnp.float32),
            jax.ShapeDtypeStruct((NASSIGN,), jnp.int32),
        ],
        mesh=plsc.VectorSubcoreMesh(**_SC_MESH),
        compiler_params=pltpu.CompilerParams(needs_layout_passes=False),
        scratch_types=[
            pltpu.VMEM((NASSIGN,), jnp.int32),
            pltpu.VMEM((NASSIGN,), jnp.int32),
            pltpu.VMEM((NASSIGN,), jnp.float32),
            pltpu.VMEM((NK,), jnp.int32),
            pltpu.VMEM((PPAD,), jnp.int32),
            pltpu.VMEM((PPAD,), jnp.float32),
            pltpu.VMEM((NASSIGN,), jnp.int32),
        ],
    )
    return fn(exp_flat, rank_flat, w_flat, offs)


def _combine_body(ys_hbm, pos_hbm, out_hbm,
                  i0_v, i1_v, r0_v, r1_v, sem0, sem1):
    wid = lax.axis_index("c") * 16 + lax.axis_index("s")
    base = wid * TOK_W

    def body(ch, carry):
        o = base + ch * 16
        pltpu.sync_copy(pos_hbm.at[pl.ds(o, 16)], i0_v)
        pltpu.sync_copy(pos_hbm.at[pl.ds(NTOK + o, 16)], i1_v)
        d0 = pltpu.async_copy(ys_hbm.at[i0_v], r0_v, sem0)
        d1 = pltpu.async_copy(ys_hbm.at[i1_v], r1_v, sem1)
        d0.wait()
        d1.wait()

        def add_body(k, c2):
            col = k * 16
            for rr in range(16):
                r0_v[rr, pl.ds(col, 16)] = (r0_v[rr, pl.ds(col, 16)]
                                            + r1_v[rr, pl.ds(col, 16)])
            return c2

        lax.fori_loop(0, D // 16, add_body, 0)
        pltpu.sync_copy(r0_v, out_hbm.at[pl.ds(o, 16)])
        return carry

    lax.fori_loop(0, TOK_W // 16, body, 0)


def _combine(ys, pos):
    fn = pl.kernel(
        _combine_body,
        out_type=[jax.ShapeDtypeStruct((NTOK, D), jnp.float32)],
        mesh=plsc.VectorSubcoreMesh(**_SC_MESH),
        compiler_params=pltpu.CompilerParams(needs_layout_passes=False),
        scratch_types=[
            pltpu.VMEM((16,), jnp.int32),
            pltpu.VMEM((16,), jnp.int32),
            pltpu.VMEM((16, D), jnp.float32),
            pltpu.VMEM((16, D), jnp.float32),
            pltpu.SemaphoreType.DMA,
            pltpu.SemaphoreType.DMA,
        ],
    )
    return fn(ys, pos)[0]


def _mlp_body(be_ref, nb_ref, x_ref, tok_ref, gu_ref, dp_ref, w_ref, out_ref):
    b = pl.program_id(0)
    nbv = nb_ref[0]

    @pl.when(b == 0)
    def _():
        out_ref[...] = jnp.zeros_like(out_ref)

    @pl.when(b < nbv)
    def _():
        tok_row = tok_ref[0]
        sub = lax.broadcasted_iota(jnp.int32, (NTOK, T), 0)
        hit = sub == tok_row
        permt = hit.astype(jnp.bfloat16)
        xb = lax.dot_general(permt, x_ref[...], (((0,), (0,)), ((), ())),
                             preferred_element_type=jnp.float32)
        xb = xb.astype(jnp.bfloat16)
        gu = gu_ref[0].astype(jnp.bfloat16)
        g_up = lax.dot_general(xb, gu, (((1,), (1,)), ((), ())),
                               preferred_element_type=jnp.float32)
        gate = g_up[:, :DFF]
        up = g_up[:, DFF:]
        h = gate * jax.nn.sigmoid(gate) * up
        dp = dp_ref[0].astype(jnp.bfloat16)
        y = lax.dot_general(h.astype(jnp.bfloat16), dp,
                            (((1,), (1,)), ((), ())),
                            preferred_element_type=jnp.float32)
        permw = jnp.where(hit, w_ref[0], 0.0).astype(jnp.bfloat16)
        delta = lax.dot_general(permw, y.astype(jnp.bfloat16),
                                (((1,), (0,)), ((), ())),
                                preferred_element_type=jnp.float32)
        out_ref[...] += delta


def _run_mlp(x_bf, tok3, gate_up_proj, down_proj, ws3, be, nbv):
    def _clamped(b, be_s, nb_s):
        return jnp.where(b < nb_s[0], b, nb_s[0] - 1)

    grid_spec = pltpu.PrefetchScalarGridSpec(
        num_scalar_prefetch=2,
        grid=(NB,),
        in_specs=[
            pl.BlockSpec((NTOK, D), lambda b, be_s, nb_s: (0, 0)),
            pl.BlockSpec((1, 1, T), lambda b, be_s, nb_s: (_clamped(b, be_s, nb_s), 0, 0)),
            pl.BlockSpec((1, 2 * DFF, D),
                         lambda b, be_s, nb_s: (be_s[_clamped(b, be_s, nb_s)], 0, 0)),
            pl.BlockSpec((1, D, DFF),
                         lambda b, be_s, nb_s: (be_s[_clamped(b, be_s, nb_s)], 0, 0)),
            pl.BlockSpec((1, 1, T), lambda b, be_s, nb_s: (_clamped(b, be_s, nb_s), 0, 0)),
        ],
        out_specs=pl.BlockSpec((NTOK, D), lambda b, be_s, nb_s: (0, 0)),
    )
    return pl.pallas_call(
        _mlp_body,
        grid_spec=grid_spec,
        out_shape=jax.ShapeDtypeStruct((NTOK, D), jnp.float32),
        compiler_params=pltpu.CompilerParams(
            vmem_limit_bytes=120 * 1024 * 1024),
        interpret=False,
    )(be, nbv, x_bf, tok3, gate_up_proj, down_proj, ws3)


def kernel(hidden_states, gate_weight, gate_up_proj, down_proj, old_to_new):
    bsz, seq, _ = hidden_states.shape
    x2d = hidden_states.reshape(NTOK, D)

    gkp = jnp.zeros((EPAD, D), jnp.float32).at[:NE].set(gate_weight)
    pruned = old_to_new < 0
    bias_row = jnp.where(
        jnp.concatenate([pruned, jnp.ones((EPAD - NE,), bool)]),
        jnp.float32(NEG), jnp.float32(0.0)).reshape(1, EPAD)
    o2n_row = jnp.concatenate(
        [old_to_new, jnp.zeros((EPAD - NE,), jnp.int32)]).reshape(1, EPAD)

    exp_flat, rank_flat, w_flat, offs, be, nbv8 = _run_router(
        x2d, gkp, bias_row, o2n_row)

    tok_sorted, w_sorted, pos = _scatter_meta(exp_flat, rank_flat, w_flat, offs)

    x_bf = x2d.astype(jnp.bfloat16)
    tok3 = tok_sorted.reshape(NB, 1, T)
    ws3 = w_sorted.reshape(NB, 1, T)
    out = _run_mlp(x_bf, tok3, gate_up_proj, down_proj, ws3, be, nbv8[:1])
    return out.reshape(bsz, seq, D)
